# baseline (speedup 1.0000x reference)
import jax
import jax.numpy as jnp
from jax.experimental import pallas as pl
from jax.experimental.pallas import tpu as pltpu

CAP = 1_000_000
BLK = 8192
NB = (CAP + BLK - 1) // BLK
NEG = -3.0e38


def _body(idx_ref, phs_ref, ph_ref, trajT_ref, bufT_ref, out_ref,
          rmax_ref, rbid_ref, rrow_ref):
    b = pl.program_id(0)

    @pl.when(b == 0)
    def _init():
        rmax_ref[...] = jnp.full((1, BLK), NEG, jnp.float32)
        rbid_ref[...] = jnp.zeros((1, BLK), jnp.int32)
        rrow_ref[...] = jnp.zeros((8, BLK), jnp.float32)

    blk = bufT_ref[...]
    bbf = blk[0:4, :].astype(jnp.bfloat16).astype(jnp.float32)
    prod = bbf * ph_ref[...]
    sims = (prod[0:1, :] + prod[1:2, :]) + (prod[2:3, :] + prod[3:4, :])

    col = jax.lax.broadcasted_iota(jnp.int32, (1, BLK), 1)
    idx = idx_ref[0]
    idx_local = idx - b * BLK
    limit = jnp.minimum(BLK, CAP - b * BLK)
    ok = (col != idx_local) & (col < limit)

    upd = (sims > rmax_ref[...]) & ok
    rmax_ref[...] = jnp.where(upd, sims, rmax_ref[...])
    rbid_ref[...] = jnp.where(upd, b, rbid_ref[...])
    rrow_ref[...] = jnp.where(jnp.broadcast_to(upd, (8, BLK)), blk, rrow_ref[...])

    @pl.when(b == NB - 1)
    def _finish():
        rmax = rmax_ref[...]
        gmax = jnp.max(rmax)
        gidx = rbid_ref[...] * BLK + col
        cand = jnp.where(rmax == gmax, gidx, jnp.int32(2**31 - 1))
        best = jnp.min(cand)
        jstar = best % BLK
        sel = jnp.broadcast_to(col == jstar, (8, BLK))
        roww = jnp.sum(jnp.where(sel, rrow_ref[...], 0.0), axis=1, keepdims=True)
        row_act = roww[4:7, :]

        asum = jnp.sum(trajT_ref[...], axis=1, keepdims=True)
        theta = jnp.sqrt(jnp.sum(asum * asum))
        axis = asum / (theta + 1e-8)
        qr = jnp.cos(theta)
        qi = axis * jnp.sin(theta)
        to_f = lambda x: x.astype(jnp.bfloat16).astype(jnp.float32)
        sim_e = (to_f(qr) * phs_ref[0] + to_f(qi[0, 0]) * phs_ref[1]
                 + to_f(qi[1, 0]) * phs_ref[2] + to_f(qi[2, 0]) * phs_ref[3])
        win_e = (sim_e > gmax) | ((sim_e == gmax) & (idx < best))

        res = jnp.where(win_e, asum, row_act)
        out_ref[...] = jnp.broadcast_to(res, (3, 128))


def kernel(trajectory_lie_elements, value, current_phase, buffer, ptr):
    del value
    idx = (jnp.asarray(ptr, jnp.int32) % CAP).reshape(1)
    bufT = buffer.T
    trajT = trajectory_lie_elements.T
    ph_col = current_phase.reshape(4, 1)

    out = pl.pallas_call(
        _body,
        grid=(NB,),
        in_specs=[
            pl.BlockSpec(memory_space=pltpu.SMEM),
            pl.BlockSpec(memory_space=pltpu.SMEM),
            pl.BlockSpec((4, 1), lambda b: (0, 0)),
            pl.BlockSpec((3, 8192), lambda b: (0, 0)),
            pl.BlockSpec((8, BLK), lambda b: (0, b)),
        ],
        out_specs=pl.BlockSpec((3, 128), lambda b: (0, 0)),
        out_shape=jax.ShapeDtypeStruct((3, 128), jnp.float32),
        scratch_shapes=[
            pltpu.VMEM((1, BLK), jnp.float32),
            pltpu.VMEM((1, BLK), jnp.int32),
            pltpu.VMEM((8, BLK), jnp.float32),
        ],
        compiler_params=pltpu.CompilerParams(
            dimension_semantics=("arbitrary",),
        ),
    )(idx, current_phase, ph_col, trajT, bufT)
    return out[:, 0]
